# Initial kernel scaffold; baseline (speedup 1.0000x reference)
import jax
import jax.numpy as jnp
from jax.experimental import pallas as pl


def kernel(node_feats, coords, batch_index, W_w, W_b, f1, f2, u1_w, u1_b, u2_w, u2_b):
    raise NotImplementedError("write your pallas kernel here")


# speedup vs baseline: 17.0879x; 17.0879x over previous
import jax
import jax.numpy as jnp
from jax.experimental import pallas as pl
from jax.experimental.pallas import tpu as pltpu

HIDDEN = 128
NUM_BASES = 64
RADIUS = 0.25
D_MIN = 0.0
D_MAX = 0.25

T_D = 128
T_S = 128


def _cfc_kernel(lo_ref, hi_ref, crows_ref, ccols_ref, featsT_ref,
                A_ref, B_ref, u1t_ref, u1b_ref, u2t_ref, u2b_ref,
                out_ref, acc_ref):
    i = pl.program_id(0)
    j = pl.program_id(1)
    nj = pl.num_programs(1)

    @pl.when(j == 0)
    def _init():
        acc_ref[...] = jnp.zeros_like(acc_ref)

    lo = lo_ref[i]
    hi = hi_ref[i]
    s0 = j * T_S
    active = jnp.logical_and(s0 < hi, s0 + T_S > lo)

    @pl.when(active)
    def _body():
        d0 = i * T_D
        cd = crows_ref[pl.ds(d0, T_D), :]
        cs = ccols_ref[:, pl.ds(s0, T_S)]
        d2 = jnp.zeros((T_D, T_S), jnp.float32)
        for c in range(3):
            diff = cd[:, c:c + 1] - cs[c:c + 1, :]
            d2 = d2 + diff * diff
        dist = jnp.sqrt(d2)
        bd = cd[:, 3:4]
        bs = cs[4:5, :]
        ids_d = d0 + jax.lax.broadcasted_iota(jnp.int32, (T_D, 1), 0)
        ids_s = s0 + jax.lax.broadcasted_iota(jnp.int32, (1, T_S), 1)
        mask = (dist < RADIUS) & (bd == bs) & (ids_d != ids_s)
        maskf = mask.astype(jnp.float32).reshape(T_D, 1, T_S)
        step = (D_MAX - D_MIN) / (NUM_BASES - 1)
        coeff = -0.5 / (step * step)
        offs = D_MIN + step * jax.lax.broadcasted_iota(
            jnp.int32, (1, NUM_BASES, 1), 1).astype(jnp.float32)
        dd = dist.reshape(T_D, 1, T_S) - offs
        rbf = jnp.exp(coeff * (dd * dd)) * maskf
        A3 = jnp.broadcast_to(A_ref[...][None], (T_D, NUM_BASES, HIDDEN))
        B3 = jnp.broadcast_to(B_ref[...][None], (T_D, HIDDEN, HIDDEN))
        m1 = jnp.maximum(jax.lax.dot_general(
            A3, rbf, (((1,), (1,)), ((0,), (0,))),
            preferred_element_type=jnp.float32), 0.0)
        m2 = jnp.maximum(jax.lax.dot_general(
            B3, m1, (((1,), (1,)), ((0,), (0,))),
            preferred_element_type=jnp.float32), 0.0)
        fT = featsT_ref[:, pl.ds(s0, T_S)]
        acc_ref[...] += jnp.sum(m2 * fT.reshape(1, HIDDEN, T_S), axis=2)

    @pl.when(j == nj - 1)
    def _epilogue():
        h = acc_ref[...]
        t = jnp.maximum(
            jnp.dot(h, u1t_ref[...], preferred_element_type=jnp.float32)
            + u1b_ref[...], 0.0)
        out_ref[...] = (jnp.dot(t, u2t_ref[...],
                                preferred_element_type=jnp.float32)
                        + u2b_ref[...])


def kernel(node_feats, coords, batch_index, W_w, W_b, f1, f2,
           u1_w, u1_b, u2_w, u2_b):
    n, hidden = node_feats.shape
    n_pad = ((n + T_D - 1) // T_D) * T_D
    pad = n_pad - n
    n_ti = n_pad // T_D
    n_tj = n_pad // T_S

    featsT = jnp.pad(node_feats, ((0, pad), (0, 0))).T
    bi = batch_index.astype(jnp.float32)
    b_dst = jnp.pad(bi, (0, pad), constant_values=-1.0)
    b_src = jnp.pad(bi, (0, pad), constant_values=-2.0)
    cpad = jnp.pad(coords, ((0, pad), (0, 0)))
    zeros = jnp.zeros((n_pad,), jnp.float32)
    crows = jnp.stack([cpad[:, 0], cpad[:, 1], cpad[:, 2], b_dst,
                       b_src, zeros, zeros, zeros], axis=1)
    ccols = crows.T

    t_ids = jnp.arange(n_ti, dtype=jnp.int32)
    first_idx = jnp.minimum(t_ids * T_D, n - 1)
    last_idx = jnp.minimum(t_ids * T_D + (T_D - 1), n - 1)
    g_first = batch_index[first_idx]
    g_last = batch_index[last_idx]
    lo = jnp.searchsorted(batch_index, g_first, side='left').astype(jnp.int32)
    hi = jnp.searchsorted(batch_index, g_last, side='right').astype(jnp.int32)

    A = f1.T
    B = f2.T
    u1t = u1_w.T
    u2t = u2_w.T
    u1b = u1_b.reshape(1, hidden)
    u2b = u2_b.reshape(1, hidden)

    smem = pl.BlockSpec(memory_space=pltpu.SMEM)

    def full(a):
        return pl.BlockSpec(a.shape, lambda i, j: (0,) * a.ndim)

    out = pl.pallas_call(
        _cfc_kernel,
        grid=(n_ti, n_tj),
        in_specs=[smem, smem,
                  full(crows), full(ccols), full(featsT),
                  full(A), full(B), full(u1t), full(u1b), full(u2t),
                  full(u2b)],
        out_specs=pl.BlockSpec((T_D, hidden), lambda i, j: (i, 0)),
        out_shape=jax.ShapeDtypeStruct((n_pad, hidden), jnp.float32),
        scratch_shapes=[pltpu.VMEM((T_D, hidden), jnp.float32)],
        compiler_params=pltpu.CompilerParams(
            dimension_semantics=("arbitrary", "arbitrary")),
    )(lo, hi, crows, ccols, featsT, A, B, u1t, u1b, u2t, u2b)
    return out[:n]


# speedup vs baseline: 17.8298x; 1.0434x over previous
import jax
import jax.numpy as jnp
from jax.experimental import pallas as pl
from jax.experimental.pallas import tpu as pltpu

HIDDEN = 128
NUM_BASES = 64
RADIUS = 0.25
D_MIN = 0.0
D_MAX = 0.25

T_D = 128
T_S = 128


def _cfc_kernel(lo_ref, hi_ref, crows_ref, ccols_ref, featsT_ref,
                A_ref, B_ref, u1t_ref, u1b_ref, u2t_ref, u2b_ref,
                out_ref, acc_ref):
    i = pl.program_id(0)
    j = pl.program_id(1)
    nj = pl.num_programs(1)

    @pl.when(j == 0)
    def _init():
        acc_ref[...] = jnp.zeros_like(acc_ref)

    lo = lo_ref[i]
    hi = hi_ref[i]
    s0 = j * T_S
    active = jnp.logical_and(s0 < hi, s0 + T_S > lo)

    @pl.when(active)
    def _body():
        d0 = i * T_D
        cd = crows_ref[pl.ds(d0, T_D), :]
        cs = ccols_ref[:, pl.ds(s0, T_S)]
        d2 = jnp.zeros((T_D, T_S), jnp.float32)
        for c in range(3):
            diff = cd[:, c:c + 1] - cs[c:c + 1, :]
            d2 = d2 + diff * diff
        dist = jnp.sqrt(d2)
        bd = cd[:, 3:4]
        bs = cs[4:5, :]
        ids_d = d0 + jax.lax.broadcasted_iota(jnp.int32, (T_D, 1), 0)
        ids_s = s0 + jax.lax.broadcasted_iota(jnp.int32, (1, T_S), 1)
        mask = (dist < RADIUS) & (bd == bs) & (ids_d != ids_s)
        maskf = mask.astype(jnp.float32).reshape(T_D, 1, T_S)
        step = (D_MAX - D_MIN) / (NUM_BASES - 1)
        coeff = -0.5 / (step * step)
        offs = D_MIN + step * jax.lax.broadcasted_iota(
            jnp.int32, (1, NUM_BASES, 1), 1).astype(jnp.float32)
        dd = dist.reshape(T_D, 1, T_S) - offs
        rbf = jnp.exp(coeff * (dd * dd)) * maskf
        A3 = jnp.broadcast_to(A_ref[...].astype(jnp.bfloat16)[None],
                              (T_D, NUM_BASES, HIDDEN))
        B3 = jnp.broadcast_to(B_ref[...].astype(jnp.bfloat16)[None],
                              (T_D, HIDDEN, HIDDEN))
        m1 = jnp.maximum(jax.lax.dot_general(
            A3, rbf.astype(jnp.bfloat16), (((1,), (1,)), ((0,), (0,))),
            preferred_element_type=jnp.float32), 0.0)
        m2 = jnp.maximum(jax.lax.dot_general(
            B3, m1.astype(jnp.bfloat16), (((1,), (1,)), ((0,), (0,))),
            preferred_element_type=jnp.float32), 0.0)
        fT = featsT_ref[:, pl.ds(s0, T_S)]
        acc_ref[...] += jnp.sum(m2 * fT.reshape(1, HIDDEN, T_S), axis=2)

    @pl.when(j == nj - 1)
    def _epilogue():
        h = acc_ref[...]
        t = jnp.maximum(
            jnp.dot(h, u1t_ref[...], preferred_element_type=jnp.float32)
            + u1b_ref[...], 0.0)
        out_ref[...] = (jnp.dot(t, u2t_ref[...],
                                preferred_element_type=jnp.float32)
                        + u2b_ref[...])


def kernel(node_feats, coords, batch_index, W_w, W_b, f1, f2,
           u1_w, u1_b, u2_w, u2_b):
    n, hidden = node_feats.shape
    n_pad = ((n + T_D - 1) // T_D) * T_D
    pad = n_pad - n
    n_ti = n_pad // T_D
    n_tj = n_pad // T_S

    featsT = jnp.pad(node_feats, ((0, pad), (0, 0))).T
    bi = batch_index.astype(jnp.float32)
    b_dst = jnp.pad(bi, (0, pad), constant_values=-1.0)
    b_src = jnp.pad(bi, (0, pad), constant_values=-2.0)
    cpad = jnp.pad(coords, ((0, pad), (0, 0)))
    zeros = jnp.zeros((n_pad,), jnp.float32)
    crows = jnp.stack([cpad[:, 0], cpad[:, 1], cpad[:, 2], b_dst,
                       b_src, zeros, zeros, zeros], axis=1)
    ccols = crows.T

    t_ids = jnp.arange(n_ti, dtype=jnp.int32)
    first_idx = jnp.minimum(t_ids * T_D, n - 1)
    last_idx = jnp.minimum(t_ids * T_D + (T_D - 1), n - 1)
    g_first = batch_index[first_idx]
    g_last = batch_index[last_idx]
    lo = jnp.searchsorted(batch_index, g_first, side='left').astype(jnp.int32)
    hi = jnp.searchsorted(batch_index, g_last, side='right').astype(jnp.int32)

    A = f1.T
    B = f2.T
    u1t = u1_w.T
    u2t = u2_w.T
    u1b = u1_b.reshape(1, hidden)
    u2b = u2_b.reshape(1, hidden)

    smem = pl.BlockSpec(memory_space=pltpu.SMEM)

    def full(a):
        return pl.BlockSpec(a.shape, lambda i, j: (0,) * a.ndim)

    out = pl.pallas_call(
        _cfc_kernel,
        grid=(n_ti, n_tj),
        in_specs=[smem, smem,
                  full(crows), full(ccols), full(featsT),
                  full(A), full(B), full(u1t), full(u1b), full(u2t),
                  full(u2b)],
        out_specs=pl.BlockSpec((T_D, hidden), lambda i, j: (i, 0)),
        out_shape=jax.ShapeDtypeStruct((n_pad, hidden), jnp.float32),
        scratch_shapes=[pltpu.VMEM((T_D, hidden), jnp.float32)],
        compiler_params=pltpu.CompilerParams(
            dimension_semantics=("arbitrary", "arbitrary")),
    )(lo, hi, crows, ccols, featsT, A, B, u1t, u1b, u2t, u2b)
    return out[:n]
